# baseline (speedup 1.0000x reference)
import functools

import jax
import jax.numpy as jnp
from jax import lax
from jax.experimental import pallas as pl
from jax.experimental.pallas import tpu as pltpu
from jax.experimental.pallas import tpu_sc as plsc

N = 1_000_000
D = 64
U = 16_384
NW = 32
BLK = 128
NBF = N // BLK
TAIL0 = NBF * BLK
TAILW = N - TAIL0
BPW = NBF // NW
BXT = NBF - BPW * NW
UV = U // 16
CAP = U + 16
SENTINEL = 0x7FFFFFFF

_LANE = lambda: lax.iota(jnp.int32, 16)


def _vshift_down(v):
    idx = jnp.minimum(_LANE() + 1, 15)
    return jnp.take_along_axis(v, idx, axis=0, mode="promise_in_bounds")


def _sc_body(xT_hbm, idx_hbm, updP_hbm, out_hbm,
             idx_v, pos_list, union2d, win_pos, win_col,
             xblk0, xblk1, blist16, blist_rel,
             in_sem0, in_sem1, out_sem0, out_sem1, g_sem):
    wid = lax.axis_index("s") * 2 + lax.axis_index("c")
    b_lo = BPW * wid + jnp.minimum(wid, BXT)
    b_hi = BPW * (wid + 1) + jnp.minimum(wid + 1, BXT)
    nb = b_hi - b_lo
    lo = b_lo * BLK
    hi = b_hi * BLK

    pltpu.sync_copy(idx_hbm, idx_v)

    lane = _LANE()
    ones = jnp.ones((16,), jnp.int32)
    lo_v = jnp.full((16,), lo, jnp.int32)
    span_u32 = (hi - lo).astype(jnp.uint32)

    def scan_body(i, off_v):
        v = idx_v[pl.ds(i * 16, 16)]
        rel = v - lo_v
        mask = rel.astype(jnp.uint32) < span_u32
        cs = plsc.cumsum(ones, mask=mask)
        dest = off_v + cs - 1
        pos = i * 16 + lane
        plsc.store_scatter(pos_list, [dest], pos, mask=mask)
        return off_v + plsc.all_reduce_population_count(mask)

    off_v = lax.fori_loop(0, UV, scan_body, jnp.zeros((16,), jnp.int32))
    m = jnp.max(off_v)
    mw = (m + 15) // 16
    m_splat = jnp.full((16,), m, jnp.int32)

    def tag_body(j, _):
        ent = j * 16 + lane
        valid = ent < m_splat
        p = plsc.load_gather(pos_list, [jnp.where(valid, ent, 0)])
        v = plsc.load_gather(idx_v, [p], mask=valid)
        rel = v - lo_v
        combined = jnp.where(valid, rel * 16 + lane, SENTINEL)
        sk, _sv = plsc.sort_key_val(combined, combined)
        rel_s = lax.shift_right_logical(sk, 4)
        ent_s = j * 16 + (sk & 15)
        rel_n = lax.shift_right_logical(_vshift_down(sk), 4)
        keep = ((rel_s != rel_n) | (lane == 15)) & (sk != SENTINEL)
        plsc.store_scatter(
            union2d,
            [lax.shift_right_logical(rel_s, 7), rel_s & (BLK - 1)],
            plsc.bitcast(ent_s, jnp.float32), mask=keep)
        return 0

    lax.fori_loop(0, mw, tag_body, 0)

    def win_body(j, off2_v):
        ent = j * 16 + lane
        valid = ent < m_splat
        p = plsc.load_gather(pos_list, [jnp.where(valid, ent, 0)])
        v = plsc.load_gather(idx_v, [p], mask=valid)
        rel = jnp.where(valid, v - lo_v, 0)
        t = plsc.bitcast(
            plsc.load_gather(
                union2d,
                [lax.shift_right_logical(rel, 7), rel & (BLK - 1)]),
            jnp.int32)
        win = valid & (t == ent)
        cs = plsc.cumsum(ones, mask=win)
        dest = off2_v + cs - 1
        plsc.store_scatter(win_pos, [dest], p, mask=win)
        plsc.store_scatter(win_col, [dest], v, mask=win)
        return off2_v + plsc.all_reduce_population_count(win)

    off2_v = lax.fori_loop(0, mw, win_body, jnp.zeros((16,), jnp.int32))
    mf = jnp.max(off2_v)
    mfw = (mf + 15) // 16
    mf_splat = jnp.full((16,), mf, jnp.int32)

    def _in_dma(b, xref, sem):
        return pltpu.make_async_copy(
            xT_hbm.at[:, pl.ds(b * BLK, BLK)], xref, sem)

    def _out_dma(b, xref, sem):
        return pltpu.make_async_copy(
            xref, out_hbm.at[:, pl.ds(b * BLK, BLK)], sem)

    def _process(base_col, width, xref):
        base_v = jnp.full((16,), base_col, jnp.int32)
        width_u32 = jnp.uint32(width)
        zeros16 = jnp.zeros((16,), jnp.int32)
        for g in range(8):
            blist16[g, :] = zeros16

        def bscan(j, k_v):
            ent = j * 16 + lane
            valid = ent < mf_splat
            col = plsc.load_gather(win_col, [jnp.where(valid, ent, 0)])
            rel = col - base_v
            bmask = valid & (rel.astype(jnp.uint32) < width_u32)
            p = plsc.load_gather(win_pos, [jnp.where(valid, ent, 0)])
            cs = plsc.cumsum(ones, mask=bmask)
            dest = k_v + cs - 1
            plsc.store_scatter(
                blist16, [lax.shift_right_logical(dest, 4), dest & 15],
                p, mask=bmask)
            plsc.store_scatter(blist_rel, [dest], rel, mask=bmask)
            return k_v + plsc.all_reduce_population_count(bmask)

        k_v = lax.fori_loop(0, mfw, bscan, jnp.zeros((16,), jnp.int32))
        kb = jnp.max(k_v)

        @pl.when(kb > 0)
        def _apply():
            ng = (kb + 15) // 16

            def fetch(g, _):
                pltpu.async_copy(
                    updP_hbm.at[blist16.at[g]],
                    union2d.at[pl.ds(g * 16, 16)], g_sem).wait()
                return 0

            lax.fori_loop(0, ng, fetch, 0)

            def apply_vreg(j2, _):
                s = j2 * 16 + lane
                vmask = s < jnp.full((16,), kb, jnp.int32)
                rel = plsc.load_gather(
                    blist_rel, [jnp.where(vmask, s, 0)])
                for c in range(D):
                    val = plsc.load_gather(
                        union2d, [s, jnp.full((16,), c, jnp.int32)],
                        mask=vmask)
                    plsc.store_scatter(
                        xref, [jnp.full((16,), c, jnp.int32), rel],
                        val, mask=vmask)
                return 0

            lax.fori_loop(0, ng, apply_vreg, 0)

    def _slot_process(i, b):
        @pl.when(i % 2 == 0)
        def _():
            _process(b * BLK, BLK, xblk0)

        @pl.when(i % 2 == 1)
        def _():
            _process(b * BLK, BLK, xblk1)

    _in_dma(b_lo, xblk0, in_sem0).start()

    def block_body(i, _):
        b = b_lo + i
        even = i % 2 == 0

        @pl.when(even)
        def _():
            _in_dma(b, xblk0, in_sem0).wait()

        @pl.when(jnp.logical_not(even))
        def _():
            _in_dma(b, xblk1, in_sem1).wait()

        _slot_process(i, b)

        @pl.when(even)
        def _():
            _out_dma(b, xblk0, out_sem0).start()

        @pl.when(jnp.logical_not(even))
        def _():
            _out_dma(b, xblk1, out_sem1).start()

        @pl.when(i + 1 < nb)
        def _():
            @pl.when((i >= 1) & jnp.logical_not(even))
            def _():
                _out_dma(b - 1, xblk0, out_sem0).wait()

            @pl.when((i >= 1) & even)
            def _():
                _out_dma(b - 1, xblk1, out_sem1).wait()

            @pl.when(even)
            def _():
                _in_dma(b + 1, xblk1, in_sem1).start()

            @pl.when(jnp.logical_not(even))
            def _():
                _in_dma(b + 1, xblk0, in_sem0).start()

        return 0

    lax.fori_loop(0, nb, block_body, 0)

    @pl.when(nb > 1)
    def _():
        last2 = b_hi - 2
        i2 = nb - 2

        @pl.when(i2 % 2 == 0)
        def _():
            _out_dma(last2, xblk0, out_sem0).wait()

        @pl.when(i2 % 2 == 1)
        def _():
            _out_dma(last2, xblk1, out_sem1).wait()

    last1 = b_hi - 1
    i1 = nb - 1

    @pl.when(i1 % 2 == 0)
    def _():
        _out_dma(last1, xblk0, out_sem0).wait()

    @pl.when(i1 % 2 == 1)
    def _():
        _out_dma(last1, xblk1, out_sem1).wait()


@jax.jit
def kernel(x, indices, updates):
    xT = x.T
    idx_flat = indices.reshape(U)
    updP = jnp.pad(updates, ((0, 0), (0, BLK - D)))
    mesh = plsc.VectorSubcoreMesh(core_axis_name="c", subcore_axis_name="s")
    scatter = pl.kernel(
        _sc_body,
        out_type=jax.ShapeDtypeStruct((D, N), jnp.float32),
        mesh=mesh,
        compiler_params=pltpu.CompilerParams(needs_layout_passes=False),
        scratch_types=[
            pltpu.VMEM((U,), jnp.int32),
            pltpu.VMEM((CAP,), jnp.int32),
            pltpu.VMEM((256, BLK), jnp.float32),
            pltpu.VMEM((CAP,), jnp.int32),
            pltpu.VMEM((CAP,), jnp.int32),
            pltpu.VMEM((D, BLK), jnp.float32),
            pltpu.VMEM((D, BLK), jnp.float32),
            pltpu.VMEM((8, 16), jnp.int32),
            pltpu.VMEM((BLK,), jnp.int32),
            pltpu.SemaphoreType.DMA,
            pltpu.SemaphoreType.DMA,
            pltpu.SemaphoreType.DMA,
            pltpu.SemaphoreType.DMA,
            pltpu.SemaphoreType.DMA,
        ],
    )
    outT = scatter(xT, idx_flat, updP)
    out = outT.T

    tail_ids = TAIL0 + jnp.arange(TAILW, dtype=jnp.int32)
    cmp = idx_flat[None, :] == tail_ids[:, None]
    pos = jnp.where(cmp, jnp.arange(U, dtype=jnp.int32)[None, :], -1)
    w = pos.max(axis=1)
    gathered = updates[jnp.clip(w, 0), :]
    tail_res = jnp.where((w >= 0)[:, None],
                         gathered, lax.slice(x, (TAIL0, 0), (N, D)))
    return lax.dynamic_update_slice(out, tail_res, (TAIL0, 0))


# speedup vs baseline: 1.5781x; 1.5781x over previous
import jax
import jax.numpy as jnp
from jax import lax
from jax.experimental import pallas as pl
from jax.experimental.pallas import tpu as pltpu
from jax.experimental.pallas import tpu_sc as plsc

N = 1_000_000
D = 64
U = 16_384
NW = 32
BLK = 256
NBF = N // BLK
TAIL0 = NBF * BLK
TAILW = N - TAIL0
BPW = NBF // NW
BXT = NBF - BPW * NW
UV = U // 16
SENTINEL = 0x7FFFFFFF

_LANE = lambda: lax.iota(jnp.int32, 16)


def _vshift_down(v):
    idx = jnp.minimum(_LANE() + 1, 15)
    return jnp.take_along_axis(v, idx, axis=0, mode="promise_in_bounds")


def _sc_body(xT_hbm, idx2d_hbm, updP_hbm, out_hbm,
             xb0, xb1, xb2, union2d, win_pos, win_col, blist16, blist_rel,
             in_sem0, in_sem1, in_sem2, out_sem0, out_sem1, out_sem2, g_sem):
    wid = lax.axis_index("s") * 2 + lax.axis_index("c")
    b_lo = BPW * wid + jnp.minimum(wid, BXT)
    b_hi = BPW * (wid + 1) + jnp.minimum(wid + 1, BXT)
    nb = b_hi - b_lo
    lo = b_lo * BLK
    hi = b_hi * BLK

    XBS = (xb0, xb1, xb2)
    IN_SEMS = (in_sem0, in_sem1, in_sem2)
    OUT_SEMS = (out_sem0, out_sem1, out_sem2)

    def _with_slot(s_dyn, fn):
        for s in range(3):
            @pl.when(s_dyn == s)
            def _(s=s):
                fn(s)

    pltpu.sync_copy(idx2d_hbm, xb0)

    lane = _LANE()
    ones = jnp.ones((16,), jnp.int32)
    zero_v = jnp.zeros((16,), jnp.int32)
    lo_v = jnp.full((16,), lo, jnp.int32)
    span_u32 = (hi - lo).astype(jnp.uint32)

    def _idx_gather(p):
        return plsc.load_gather(
            xb0, [lax.shift_right_logical(p, 8), p & 255])

    def _pos_gather(ent):
        return plsc.load_gather(
            xb1, [lax.shift_right_logical(ent, 8), ent & 255])

    def scan_body(i, off_v):
        pos = i * 16 + lane
        v = _idx_gather(pos)
        rel = v - lo_v
        mask = rel.astype(jnp.uint32) < span_u32
        cs = plsc.cumsum(ones, mask=mask)
        dest = off_v + cs - 1
        plsc.store_scatter(
            xb1, [lax.shift_right_logical(dest, 8), dest & 255],
            pos, mask=mask)
        return off_v + plsc.all_reduce_population_count(mask)

    off_v = lax.fori_loop(0, UV, scan_body, jnp.zeros((16,), jnp.int32))
    m = jnp.max(off_v)
    mw = (m + 15) // 16
    m_splat = jnp.full((16,), m, jnp.int32)

    def tag_body(j, _):
        ent = j * 16 + lane
        valid = ent < m_splat
        p = _pos_gather(jnp.where(valid, ent, 0))
        v = _idx_gather(jnp.where(valid, p, 0))
        rel = v - lo_v
        combined = jnp.where(valid, rel * 16 + lane, SENTINEL)
        sk, _sv = plsc.sort_key_val(combined, combined)
        rel_s = lax.shift_right_logical(sk, 4)
        ent_s = j * 16 + (sk & 15)
        rel_n = lax.shift_right_logical(_vshift_down(sk), 4)
        keep = ((rel_s != rel_n) | (lane == 15)) & (sk != SENTINEL)
        plsc.store_scatter(
            union2d,
            [lax.shift_right_logical(rel_s, 7), rel_s & 127],
            ent_s, mask=keep)
        return 0

    lax.fori_loop(0, mw, tag_body, 0)

    def win_body(j, off2_v):
        ent = j * 16 + lane
        valid = ent < m_splat
        p = _pos_gather(jnp.where(valid, ent, 0))
        v = _idx_gather(jnp.where(valid, p, 0))
        rel = jnp.where(valid, v - lo_v, 0)
        t = plsc.load_gather(
            union2d, [lax.shift_right_logical(rel, 7), rel & 127])
        win = valid & (t == ent)
        cs = plsc.cumsum(ones, mask=win)
        dest = off2_v + cs - 1
        plsc.store_scatter(win_pos, [dest], p, mask=win)
        plsc.store_scatter(win_col, [dest], v, mask=win)
        return off2_v + plsc.all_reduce_population_count(win)

    off2_v = lax.fori_loop(0, mw, win_body, jnp.zeros((16,), jnp.int32))
    mf = jnp.max(off2_v)
    mfw = (mf + 15) // 16
    mf_splat = jnp.full((16,), mf, jnp.int32)

    def _in_dma(b, s):
        return pltpu.make_async_copy(
            xT_hbm.at[:, pl.ds(b * BLK, BLK)], XBS[s], IN_SEMS[s])

    def _out_dma(b, s):
        return pltpu.make_async_copy(
            XBS[s], out_hbm.at[:, pl.ds(b * BLK, BLK)], OUT_SEMS[s])

    def _process(base_col, width, xref):
        base_v = jnp.full((16,), base_col, jnp.int32)
        width_u32 = jnp.uint32(width)
        for g in range(16):
            blist16[g, :] = zero_v

        def bscan(j, k_v):
            ent = j * 16 + lane
            valid = ent < mf_splat
            col = plsc.load_gather(win_col, [jnp.where(valid, ent, 0)])
            rel = col - base_v
            bmask = valid & (rel.astype(jnp.uint32) < width_u32)
            p = plsc.load_gather(win_pos, [jnp.where(valid, ent, 0)])
            cs = plsc.cumsum(ones, mask=bmask)
            dest = k_v + cs - 1
            plsc.store_scatter(
                blist16, [lax.shift_right_logical(dest, 4), dest & 15],
                p, mask=bmask)
            plsc.store_scatter(blist_rel, [dest], rel, mask=bmask)
            return k_v + plsc.all_reduce_population_count(bmask)

        k_v = lax.fori_loop(0, mfw, bscan, jnp.zeros((16,), jnp.int32))
        kb = jnp.max(k_v)

        @pl.when(kb > 0)
        def _apply():
            ng = (kb + 15) // 16

            def fetch(g, _):
                pltpu.async_copy(
                    updP_hbm.at[blist16.at[g]],
                    union2d.at[pl.ds(g * 16, 16)], g_sem).wait()
                return 0

            lax.fori_loop(0, ng, fetch, 0)

            def apply_vreg(j2, _):
                s = j2 * 16 + lane
                vmask = s < jnp.full((16,), kb, jnp.int32)
                rel = plsc.load_gather(
                    blist_rel, [jnp.where(vmask, s, 0)])
                for c in range(D):
                    val = plsc.load_gather(
                        union2d, [s, jnp.full((16,), c, jnp.int32)],
                        mask=vmask)
                    plsc.store_scatter(
                        xref, [jnp.full((16,), c, jnp.int32), rel],
                        val, mask=vmask)
                return 0

            lax.fori_loop(0, ng, apply_vreg, 0)

    _with_slot(b_lo % 3, lambda s: _in_dma(b_lo, s).start())
    _with_slot((b_lo + 1) % 3, lambda s: _in_dma(b_lo + 1, s).start())

    def block_body(i, _):
        b = b_lo + i
        s_dyn = b % 3

        def work(s):
            _in_dma(b, s).wait()
            _process(b * BLK, BLK, XBS[s])
            _out_dma(b, s).start()

        _with_slot(s_dyn, work)

        @pl.when(i + 2 < nb)
        def _():
            s2_dyn = (b + 2) % 3

            def prefetch(s2):
                @pl.when(i >= 1)
                def _():
                    _out_dma(b - 1, s2).wait()

                _in_dma(b + 2, s2).start()

            _with_slot(s2_dyn, prefetch)

        return 0

    lax.fori_loop(0, nb, block_body, 0)

    _with_slot((b_hi - 3) % 3, lambda s: _out_dma(b_hi - 3, s).wait())
    _with_slot((b_hi - 2) % 3, lambda s: _out_dma(b_hi - 2, s).wait())
    _with_slot((b_hi - 1) % 3, lambda s: _out_dma(b_hi - 1, s).wait())


@jax.jit
def kernel(x, indices, updates):
    xTi = lax.bitcast_convert_type(x.T, jnp.int32)
    idx_flat = indices.reshape(U)
    idx2d = idx_flat.reshape(64, 256)
    updP = lax.bitcast_convert_type(
        jnp.pad(updates, ((0, 0), (0, 128 - D))), jnp.int32)
    mesh = plsc.VectorSubcoreMesh(core_axis_name="c", subcore_axis_name="s")
    scatter = pl.kernel(
        _sc_body,
        out_type=jax.ShapeDtypeStruct((D, N), jnp.int32),
        mesh=mesh,
        compiler_params=pltpu.CompilerParams(needs_layout_passes=False),
        scratch_types=[
            pltpu.VMEM((D, BLK), jnp.int32),
            pltpu.VMEM((D, BLK), jnp.int32),
            pltpu.VMEM((D, BLK), jnp.int32),
            pltpu.VMEM((256, 128), jnp.int32),
            pltpu.VMEM((U,), jnp.int32),
            pltpu.VMEM((U,), jnp.int32),
            pltpu.VMEM((16, 16), jnp.int32),
            pltpu.VMEM((BLK,), jnp.int32),
            pltpu.SemaphoreType.DMA,
            pltpu.SemaphoreType.DMA,
            pltpu.SemaphoreType.DMA,
            pltpu.SemaphoreType.DMA,
            pltpu.SemaphoreType.DMA,
            pltpu.SemaphoreType.DMA,
            pltpu.SemaphoreType.DMA,
        ],
    )
    outT = scatter(xTi, idx2d, updP)
    out = lax.bitcast_convert_type(outT, jnp.float32).T

    tail_ids = TAIL0 + jnp.arange(TAILW, dtype=jnp.int32)
    cmp = idx_flat[None, :] == tail_ids[:, None]
    pos = jnp.where(cmp, jnp.arange(U, dtype=jnp.int32)[None, :], -1)
    w = pos.max(axis=1)
    gathered = updates[jnp.clip(w, 0), :]
    tail_res = jnp.where((w >= 0)[:, None],
                         gathered, lax.slice(x, (TAIL0, 0), (N, D)))
    return lax.dynamic_update_slice(out, tail_res, (TAIL0, 0))
